# baseline (speedup 1.0000x reference)
import functools

import jax
import jax.numpy as jnp
from jax import lax
from jax.experimental import pallas as pl
from jax.experimental.pallas import tpu as pltpu
from jax.experimental.pallas import tpu_sc as plsc

N = 10000
E = 160000
D = 256
H = 256

NC = 2
NS = 16
NW = NC * NS
LANES = 16
NPAD = 10240
TPW = NPAD // NW
BLK = 1600
NBLK = E // BLK
CHUNK = 48
GRP = CHUNK // LANES
STAGE = 1712


def _sc_body(x_hbm, src_hbm, dst_hbm, w_hbm, out_hbm,
             in_src, in_dst, in_w, cidx_v, cw_v, cd_v, rows_v, acc_v,
             semB, semS, semA):
    c = lax.axis_index("c")
    s = lax.axis_index("s")
    wid = s * NC + c
    base = wid * TPW

    def issue_block(b, buf):
        eb = pl.multiple_of(b * BLK, 8)
        bo = pl.multiple_of(buf * BLK, 8)
        pltpu.async_copy(src_hbm.at[pl.ds(eb, BLK)],
                         in_src.at[pl.ds(bo, BLK)], semB)
        pltpu.async_copy(dst_hbm.at[pl.ds(eb, BLK)],
                         in_dst.at[pl.ds(bo, BLK)], semB)
        pltpu.async_copy(w_hbm.at[pl.ds(eb, BLK)],
                         in_w.at[pl.ds(bo, BLK)], semB)

    def wait_block(b, buf):
        eb = pl.multiple_of(b * BLK, 8)
        bo = pl.multiple_of(buf * BLK, 8)
        pltpu.make_async_copy(
            src_hbm.at[pl.ds(eb, BLK)], in_src.at[pl.ds(bo, BLK)], semB).wait()
        pltpu.make_async_copy(
            dst_hbm.at[pl.ds(eb, BLK)], in_dst.at[pl.ds(bo, BLK)], semB).wait()
        pltpu.make_async_copy(
            w_hbm.at[pl.ds(eb, BLK)], in_w.at[pl.ds(bo, BLK)], semB).wait()

    def issue_gather(sb, coff, slot, sem):
        so = pl.multiple_of(sb * STAGE + coff, 16)
        ro = pl.multiple_of(slot * CHUNK, 16)
        idx = cidx_v.at[pl.ds(so, CHUNK)]
        return pltpu.async_copy(x_hbm.at[idx],
                                rows_v.at[pl.ds(ro, CHUNK)], sem)

    def wait_gather(sb, coff, slot, sem):
        so = pl.multiple_of(sb * STAGE + coff, 16)
        ro = pl.multiple_of(slot * CHUNK, 16)
        idx = cidx_v.at[pl.ds(so, CHUNK)]
        pltpu.make_async_copy(x_hbm.at[idx],
                              rows_v.at[pl.ds(ro, CHUNK)], sem).wait()

    bcast_dn = lax.GatherDimensionNumbers(
        offset_dims=(), collapsed_slice_dims=(0,), start_index_map=(0,))

    def bcast(v, j):
        idx = jnp.full((LANES, 1), j, jnp.int32)
        return lax.gather(v, idx, bcast_dn, (1,),
                          mode=lax.GatherScatterMode.PROMISE_IN_BOUNDS)

    def mul_chunk(sb, coff, slot):
        def grp_body(g, _):
            so = sb * STAGE + coff + g * LANES
            wg = cw_v[pl.ds(so, LANES)]
            dg = cd_v[pl.ds(so, LANES)]
            for j in range(LANES):
                wvec = bcast(wg, j)
                dl = dg[j]
                r = slot * CHUNK + g * LANES + j
                for k in range(D // LANES):
                    plsc.addupdate(
                        acc_v.at[dl, pl.ds(k * LANES, LANES)],
                        rows_v[r, pl.ds(k * LANES, LANES)] * wvec)
            return 0
        lax.fori_loop(0, GRP, grp_body, 0)

    def zrow(i, _):
        for j in range(D // LANES):
            acc_v[i, pl.ds(j * LANES, LANES)] = jnp.zeros((LANES,), jnp.float32)
        return 0
    lax.fori_loop(0, TPW, zrow, 0)

    issue_block(0, 0)

    def blk_body(b, carry):
        rem, nfprev = carry
        p = b & 1
        wait_block(b, p)

        @pl.when(b + 1 < NBLK)
        def _():
            issue_block(b + 1, 1 - p)

        pb = p * BLK
        ps = p * STAGE

        def scan(i, off):
            sv = in_src[pl.ds(pb + i * LANES, LANES)]
            dv = in_dst[pl.ds(pb + i * LANES, LANES)]
            wv = in_w[pl.ds(pb + i * LANES, LANES)]
            m = (dv >= base) & (dv < base + TPW)
            plsc.store_compressed(cidx_v.at[pl.ds(ps + off, LANES)], sv, mask=m)
            plsc.store_compressed(cw_v.at[pl.ds(ps + off, LANES)], wv, mask=m)
            plsc.store_compressed(cd_v.at[pl.ds(ps + off, LANES)], dv - base,
                                  mask=m)
            return off + plsc.all_reduce_population_count(m)[0]
        fill = lax.fori_loop(0, BLK // LANES, scan, rem)
        nf = fill // CHUNK

        def sync_chunk(ci, _):
            issue_gather(p, ci * CHUNK, p, semS).wait()
            mul_chunk(p, ci * CHUNK, p)
            return 0
        lax.fori_loop(0, jnp.maximum(nf - 1, 0), sync_chunk, 0)

        @pl.when(nfprev > 0)
        def _():
            wait_gather(1 - p, (nfprev - 1) * CHUNK, 1 - p, semA)
            mul_chunk(1 - p, (nfprev - 1) * CHUNK, 1 - p)

        qs = (1 - p) * STAGE
        for j in range(GRP):
            o = ps + nf * CHUNK + j * LANES
            cidx_v[pl.ds(qs + j * LANES, LANES)] = cidx_v[pl.ds(o, LANES)]
            cw_v[pl.ds(qs + j * LANES, LANES)] = cw_v[pl.ds(o, LANES)]
            cd_v[pl.ds(qs + j * LANES, LANES)] = cd_v[pl.ds(o, LANES)]

        @pl.when(nf > 0)
        def _():
            issue_gather(p, (nf - 1) * CHUNK, p, semA)

        return (fill - nf * CHUNK, nf)

    rem, nflast = lax.fori_loop(0, NBLK, blk_body, (jnp.int32(0), jnp.int32(0)))

    @pl.when(nflast > 0)
    def _():
        wait_gather(1, (nflast - 1) * CHUNK, 1, semA)
        mul_chunk(1, (nflast - 1) * CHUNK, 1)

    @pl.when(rem > 0)
    def _():
        pad_idx = jnp.full((LANES,), wid * 16, jnp.int32)
        for j in range(GRP):
            cidx_v[pl.ds(rem + j * LANES, LANES)] = pad_idx
            cw_v[pl.ds(rem + j * LANES, LANES)] = jnp.zeros(
                (LANES,), jnp.float32)
            cd_v[pl.ds(rem + j * LANES, LANES)] = jnp.zeros(
                (LANES,), jnp.int32)
        issue_gather(0, 0, 0, semS).wait()
        mul_chunk(0, 0, 0)

    pltpu.sync_copy(acc_v, out_hbm.at[pl.ds(pl.multiple_of(base, 8), TPW)])


@jax.jit
def _sc_scatter(x, src, dst, ew):
    mesh = plsc.VectorSubcoreMesh(core_axis_name="c", subcore_axis_name="s")
    f = functools.partial(
        pl.kernel,
        mesh=mesh,
        compiler_params=pltpu.CompilerParams(needs_layout_passes=False),
        out_type=jax.ShapeDtypeStruct((NPAD, D), jnp.float32),
        scratch_types=[
            pltpu.VMEM((2 * BLK,), jnp.int32),
            pltpu.VMEM((2 * BLK,), jnp.int32),
            pltpu.VMEM((2 * BLK,), jnp.float32),
            pltpu.VMEM((2 * STAGE,), jnp.int32),
            pltpu.VMEM((2 * STAGE,), jnp.float32),
            pltpu.VMEM((2 * STAGE,), jnp.int32),
            pltpu.VMEM((2 * CHUNK, D), jnp.float32),
            pltpu.VMEM((TPW, D), jnp.float32),
            pltpu.SemaphoreType.DMA,
            pltpu.SemaphoreType.DMA,
            pltpu.SemaphoreType.DMA,
        ],
    )(_sc_body)
    return f(x, src, dst, ew)


def _tc_body(agg_ref, w_ref, b_ref, g_ref, beta_ref, out_ref):
    h = jnp.dot(agg_ref[...], w_ref[...],
                preferred_element_type=jnp.float32) + b_ref[...]
    mean = jnp.mean(h, axis=-1, keepdims=True)
    cen = h - mean
    var = jnp.mean(cen * cen, axis=-1, keepdims=True)
    y = cen * lax.rsqrt(var + 1e-3) * g_ref[...] + beta_ref[...]
    out_ref[...] = jnp.maximum(y, 0.0)


@jax.jit
def _tc_norm(agg, W, b, gamma, beta):
    BM = 400
    grid = (N // BM,)
    return pl.pallas_call(
        _tc_body,
        grid=grid,
        in_specs=[
            pl.BlockSpec((BM, D), lambda i: (i, 0)),
            pl.BlockSpec((D, H), lambda i: (0, 0)),
            pl.BlockSpec((1, H), lambda i: (0, 0)),
            pl.BlockSpec((1, H), lambda i: (0, 0)),
            pl.BlockSpec((1, H), lambda i: (0, 0)),
        ],
        out_specs=pl.BlockSpec((BM, H), lambda i: (i, 0)),
        out_shape=jax.ShapeDtypeStruct((N, H), jnp.float32),
    )(agg, W, b, gamma, beta)


def kernel(x, edge_index, edge_weight, W, b, gamma, beta):
    src = edge_index[0]
    dst = edge_index[1]
    agg = _sc_scatter(x, src, dst, edge_weight)[:N]
    return _tc_norm(agg, W, b.reshape(1, H), gamma.reshape(1, H),
                    beta.reshape(1, H))


# speedup vs baseline: 2.5363x; 2.5363x over previous
import functools

import jax
import jax.numpy as jnp
from jax import lax
from jax.experimental import pallas as pl
from jax.experimental.pallas import tpu as pltpu
from jax.experimental.pallas import tpu_sc as plsc

N = 10000
E = 160000
D = 256
H = 256

NC = 2
NS = 16
NW = NC * NS
LANES = 16
NPAD = 10240
TPW = NPAD // NW
BLK = 1600
NBLK = E // BLK
CHUNK = 48
GRP = CHUNK // LANES
STAGE = 1712


def _sc_body(x_hbm, src_hbm, dst_hbm, w_hbm, out_hbm,
             in_src, in_dst, in_w, cidx_v, cw_v, cd_v, rows_v, acc_v,
             semB, semS, semA):
    c = lax.axis_index("c")
    s = lax.axis_index("s")
    wid = s * NC + c
    base = wid * TPW

    def issue_block(b, buf):
        eb = pl.multiple_of(b * BLK, 8)
        bo = pl.multiple_of(buf * BLK, 8)
        pltpu.async_copy(src_hbm.at[pl.ds(eb, BLK)],
                         in_src.at[pl.ds(bo, BLK)], semB)
        pltpu.async_copy(dst_hbm.at[pl.ds(eb, BLK)],
                         in_dst.at[pl.ds(bo, BLK)], semB)
        pltpu.async_copy(w_hbm.at[pl.ds(eb, BLK)],
                         in_w.at[pl.ds(bo, BLK)], semB)

    def wait_block(b, buf):
        eb = pl.multiple_of(b * BLK, 8)
        bo = pl.multiple_of(buf * BLK, 8)
        pltpu.make_async_copy(
            src_hbm.at[pl.ds(eb, BLK)], in_src.at[pl.ds(bo, BLK)], semB).wait()
        pltpu.make_async_copy(
            dst_hbm.at[pl.ds(eb, BLK)], in_dst.at[pl.ds(bo, BLK)], semB).wait()
        pltpu.make_async_copy(
            w_hbm.at[pl.ds(eb, BLK)], in_w.at[pl.ds(bo, BLK)], semB).wait()

    def issue_gather(sb, coff, slot, sem):
        so = pl.multiple_of(sb * STAGE + coff, 16)
        ro = pl.multiple_of(slot * CHUNK, 16)
        idx = cidx_v.at[pl.ds(so, CHUNK)]
        return pltpu.async_copy(x_hbm.at[idx],
                                rows_v.at[pl.ds(ro, CHUNK)], sem)

    def wait_gather(sb, coff, slot, sem):
        so = pl.multiple_of(sb * STAGE + coff, 16)
        ro = pl.multiple_of(slot * CHUNK, 16)
        idx = cidx_v.at[pl.ds(so, CHUNK)]
        pltpu.make_async_copy(x_hbm.at[idx],
                              rows_v.at[pl.ds(ro, CHUNK)], sem).wait()

    lane = jnp.arange(LANES, dtype=jnp.int32)

    def mul_chunk(sb, coff, slot):
        so = sb * STAGE + coff
        rbase = slot * CHUNK

        @functools.partial(plsc.parallel_loop, 0, CHUNK, unroll=2)
        def _(r):
            ivec = jnp.full((LANES,), so + r, jnp.int32)
            wvec = plsc.load_gather(cw_v, [ivec])
            dvec = plsc.load_gather(cd_v, [ivec])
            av = dvec * D + lane
            rr = rbase + r
            for k in range(D // LANES):
                plsc.addupdate_scatter(
                    acc_v, [av + k * LANES],
                    rows_v[rr, pl.ds(k * LANES, LANES)] * wvec)

    def zrow(i, _):
        for j in range(D // LANES):
            acc_v[pl.ds(i * D + j * LANES, LANES)] = jnp.zeros(
                (LANES,), jnp.float32)
        return 0
    lax.fori_loop(0, TPW, zrow, 0)

    issue_block(0, 0)

    def blk_body(b, carry):
        rem, nfprev = carry
        p = b & 1
        wait_block(b, p)

        @pl.when(b + 1 < NBLK)
        def _():
            issue_block(b + 1, 1 - p)

        pb = p * BLK
        ps = p * STAGE

        def scan(i, off):
            sv = in_src[pl.ds(pb + i * LANES, LANES)]
            dv = in_dst[pl.ds(pb + i * LANES, LANES)]
            wv = in_w[pl.ds(pb + i * LANES, LANES)]
            m = (dv >= base) & (dv < base + TPW)
            plsc.store_compressed(cidx_v.at[pl.ds(ps + off, LANES)], sv, mask=m)
            plsc.store_compressed(cw_v.at[pl.ds(ps + off, LANES)], wv, mask=m)
            plsc.store_compressed(cd_v.at[pl.ds(ps + off, LANES)], dv - base,
                                  mask=m)
            return off + plsc.all_reduce_population_count(m)[0]
        fill = lax.fori_loop(0, BLK // LANES, scan, rem)
        nf = fill // CHUNK

        def sync_chunk(ci, _):
            issue_gather(p, ci * CHUNK, p, semS).wait()
            mul_chunk(p, ci * CHUNK, p)
            return 0
        lax.fori_loop(0, jnp.maximum(nf - 1, 0), sync_chunk, 0)

        @pl.when(nfprev > 0)
        def _():
            wait_gather(1 - p, (nfprev - 1) * CHUNK, 1 - p, semA)
            mul_chunk(1 - p, (nfprev - 1) * CHUNK, 1 - p)

        qs = (1 - p) * STAGE
        for j in range(GRP):
            o = ps + nf * CHUNK + j * LANES
            cidx_v[pl.ds(qs + j * LANES, LANES)] = cidx_v[pl.ds(o, LANES)]
            cw_v[pl.ds(qs + j * LANES, LANES)] = cw_v[pl.ds(o, LANES)]
            cd_v[pl.ds(qs + j * LANES, LANES)] = cd_v[pl.ds(o, LANES)]

        @pl.when(nf > 0)
        def _():
            issue_gather(p, (nf - 1) * CHUNK, p, semA)

        return (fill - nf * CHUNK, nf)

    rem, nflast = lax.fori_loop(0, NBLK, blk_body, (jnp.int32(0), jnp.int32(0)))

    @pl.when(nflast > 0)
    def _():
        wait_gather(1, (nflast - 1) * CHUNK, 1, semA)
        mul_chunk(1, (nflast - 1) * CHUNK, 1)

    @pl.when(rem > 0)
    def _():
        pad_idx = jnp.full((LANES,), wid * 16, jnp.int32)
        for j in range(GRP):
            cidx_v[pl.ds(rem + j * LANES, LANES)] = pad_idx
            cw_v[pl.ds(rem + j * LANES, LANES)] = jnp.zeros(
                (LANES,), jnp.float32)
            cd_v[pl.ds(rem + j * LANES, LANES)] = jnp.zeros(
                (LANES,), jnp.int32)
        issue_gather(0, 0, 0, semS).wait()
        mul_chunk(0, 0, 0)

    pltpu.sync_copy(
        acc_v,
        out_hbm.at[pl.ds(pl.multiple_of(base * D, 8), TPW * D)])


@jax.jit
def _sc_scatter(x, src, dst, ew):
    mesh = plsc.VectorSubcoreMesh(core_axis_name="c", subcore_axis_name="s")
    f = functools.partial(
        pl.kernel,
        mesh=mesh,
        compiler_params=pltpu.CompilerParams(needs_layout_passes=False),
        out_type=jax.ShapeDtypeStruct((NPAD * D,), jnp.float32),
        scratch_types=[
            pltpu.VMEM((2 * BLK,), jnp.int32),
            pltpu.VMEM((2 * BLK,), jnp.int32),
            pltpu.VMEM((2 * BLK,), jnp.float32),
            pltpu.VMEM((2 * STAGE,), jnp.int32),
            pltpu.VMEM((2 * STAGE,), jnp.float32),
            pltpu.VMEM((2 * STAGE,), jnp.int32),
            pltpu.VMEM((2 * CHUNK, D), jnp.float32),
            pltpu.VMEM((TPW * D,), jnp.float32),
            pltpu.SemaphoreType.DMA,
            pltpu.SemaphoreType.DMA,
            pltpu.SemaphoreType.DMA,
        ],
    )(_sc_body)
    return f(x, src, dst, ew)


def _tc_body(agg_ref, w_ref, b_ref, g_ref, beta_ref, out_ref):
    h = jnp.dot(agg_ref[...], w_ref[...],
                preferred_element_type=jnp.float32) + b_ref[...]
    mean = jnp.mean(h, axis=-1, keepdims=True)
    cen = h - mean
    var = jnp.mean(cen * cen, axis=-1, keepdims=True)
    y = cen * lax.rsqrt(var + 1e-3) * g_ref[...] + beta_ref[...]
    out_ref[...] = jnp.maximum(y, 0.0)


@jax.jit
def _tc_norm(agg, W, b, gamma, beta):
    BM = 400
    grid = (N // BM,)
    return pl.pallas_call(
        _tc_body,
        grid=grid,
        in_specs=[
            pl.BlockSpec((BM, D), lambda i: (i, 0)),
            pl.BlockSpec((D, H), lambda i: (0, 0)),
            pl.BlockSpec((1, H), lambda i: (0, 0)),
            pl.BlockSpec((1, H), lambda i: (0, 0)),
            pl.BlockSpec((1, H), lambda i: (0, 0)),
        ],
        out_specs=pl.BlockSpec((BM, H), lambda i: (i, 0)),
        out_shape=jax.ShapeDtypeStruct((N, H), jnp.float32),
    )(agg, W, b, gamma, beta)


def kernel(x, edge_index, edge_weight, W, b, gamma, beta):
    src = edge_index[0]
    dst = edge_index[1]
    agg = _sc_scatter(x, src, dst, edge_weight).reshape(NPAD, D)[:N]
    return _tc_norm(agg, W, b.reshape(1, H), gamma.reshape(1, H),
                    beta.reshape(1, H))
